# baseline (speedup 1.0000x reference)
import functools

import jax
import jax.numpy as jnp
from jax import lax
from jax.experimental import pallas as pl
from jax.experimental.pallas import tpu as pltpu
from jax.experimental.pallas import tpu_sc as plsc

N = 10000
NP = 10240
E = 160000
G = 64
C = 16
H = 512
LANE = 128
NTILES = 16
EPT = E // NTILES
EPTP = 10000
B = 80
NB = EPTP // B
NBP = 128
KH = NB // 2
RPT = NP // NTILES
ECT = E // 32
ECTP = 5120
NBC = ECTP // B
NBT = 1000


def _sc_mesh():
    return plsc.VectorSubcoreMesh(core_axis_name="c", subcore_axis_name="s")


def _make_sc_agg(nchunk):
    cpc = nchunk // 2

    def body(x3, srcr, dst2p, zrows, out,
             src_v, dst_v, rows_v, acc_s, sem):
        cid = lax.axis_index("c")
        sid = lax.axis_index("s")
        pltpu.sync_copy(srcr.at[pl.ds(sid * EPTP, EPTP)], src_v)
        pltpu.sync_copy(dst2p.at[pl.ds(sid * NBP, NBP)], dst_v)
        r0 = sid * RPT

        def run_chunk(ci):
            table = x3.at[ci]
            pltpu.sync_copy(zrows, acc_s.at[pl.ds(r0, RPT)])
            plsc.subcore_barrier()

            def step(b, carry):
                off = pl.multiple_of(b * B, B)
                pltpu.async_copy(
                    table.at[src_v.at[pl.ds(off, B)]], rows_v, sem
                ).wait()
                pltpu.sync_copy(rows_v, acc_s.at[dst_v.at[b]], add=True)
                return carry

            lax.fori_loop(0, NB, step, 0)
            plsc.subcore_barrier()
            pltpu.sync_copy(acc_s.at[pl.ds(r0, RPT)],
                            out.at[ci].at[pl.ds(r0, RPT)])
            plsc.subcore_barrier()

        @pl.when(cid == 0)
        def _():
            for ci in range(cpc):
                run_chunk(ci)

        @pl.when(cid == 1)
        def _():
            for ci in range(cpc, nchunk):
                run_chunk(ci)

    return pl.kernel(
        body,
        out_type=jax.ShapeDtypeStruct((nchunk, NP, LANE), jnp.float32),
        mesh=_sc_mesh(),
        scratch_types=[
            pltpu.VMEM((EPTP,), jnp.int32),
            pltpu.VMEM((NBP, B), jnp.int32),
            pltpu.VMEM((B, LANE), jnp.float32),
            pltpu.VMEM_SHARED((NP, LANE), jnp.float32),
            pltpu.SemaphoreType.DMA,
        ],
    )


def _make_sc_cnt():
    def body(dstc, ones_h, zrows, out, dst_v, ones_v, acc_s):
        cid = lax.axis_index("c")
        sid = lax.axis_index("s")
        r0 = sid * RPT
        w = cid * NTILES + sid
        pltpu.sync_copy(ones_h, ones_v)
        pltpu.sync_copy(dstc.at[pl.ds(w * NBC, NBC)], dst_v)
        pltpu.sync_copy(zrows, acc_s.at[pl.ds(r0, RPT)])
        plsc.subcore_barrier()

        def step(b, carry):
            pltpu.sync_copy(ones_v, acc_s.at[dst_v.at[b]], add=True)
            return carry

        lax.fori_loop(0, NBC, step, 0)
        plsc.subcore_barrier()
        pltpu.sync_copy(acc_s.at[pl.ds(r0, RPT)],
                        out.at[cid].at[pl.ds(r0, RPT)])

    return pl.kernel(
        body,
        out_type=jax.ShapeDtypeStruct((2, NP, LANE), jnp.float32),
        mesh=_sc_mesh(),
        scratch_types=[
            pltpu.VMEM((NBC, B), jnp.int32),
            pltpu.VMEM((B, LANE), jnp.float32),
            pltpu.VMEM_SHARED((NP, LANE), jnp.float32),
        ],
    )


def _make_tc_xw(nc_in):
    def body(x_ref, w_ref, b_ref, o_ref):
        cat = jnp.concatenate([x_ref[ci] for ci in range(nc_in)], axis=1)
        acc = jnp.dot(cat, w_ref[...], preferred_element_type=jnp.float32)
        acc = acc + b_ref[...]
        for co in range(H // LANE):
            o_ref[co] = acc[:, co * LANE:(co + 1) * LANE]

    return pl.pallas_call(
        body,
        grid=(N // NBT,),
        in_specs=[
            pl.BlockSpec((nc_in, NBT, LANE), lambda i: (0, i, 0)),
            pl.BlockSpec((nc_in * LANE, H), lambda i: (0, 0)),
            pl.BlockSpec((1, H), lambda i: (0, 0)),
        ],
        out_specs=pl.BlockSpec((H // LANE, NBT, LANE), lambda i: (0, i, 0)),
        out_shape=jax.ShapeDtypeStruct((H // LANE, N, LANE), jnp.float32),
    )


def _make_tc_comb(nc_in, relu):
    def body(agg_ref, xw_ref, cnt_ref, w_ref, o_ref):
        cnt = cnt_ref[0, :, 0:1] + cnt_ref[1, :, 0:1]
        inv = 1.0 / jnp.maximum(cnt, 1.0)
        cat = jnp.concatenate([agg_ref[ci] * inv for ci in range(nc_in)],
                              axis=1)
        acc = jnp.dot(cat, w_ref[...], preferred_element_type=jnp.float32)
        acc = acc + jnp.concatenate([xw_ref[ci] for ci in range(H // LANE)],
                                    axis=1)
        if relu:
            acc = jnp.maximum(acc, 0.0)
        for co in range(H // LANE):
            o_ref[co] = acc[:, co * LANE:(co + 1) * LANE]

    return pl.pallas_call(
        body,
        grid=(N // NBT,),
        in_specs=[
            pl.BlockSpec((nc_in, NBT, LANE), lambda i: (0, i, 0)),
            pl.BlockSpec((H // LANE, NBT, LANE), lambda i: (0, i, 0)),
            pl.BlockSpec((2, NBT, LANE), lambda i: (0, i, 0)),
            pl.BlockSpec((nc_in * LANE, H), lambda i: (0, 0)),
        ],
        out_specs=pl.BlockSpec((H // LANE, NBT, LANE), lambda i: (0, i, 0)),
        out_shape=jax.ShapeDtypeStruct((H // LANE, N, LANE), jnp.float32),
    )


def _make_tc_final():
    def body(h_ref, bat_ref, w1_ref, c1_ref, w2_ref, c2_ref, w3_ref, c3_ref,
             o_ref, accp, accc):
        i = pl.program_id(0)

        @pl.when(i == 0)
        def _():
            accp[...] = jnp.zeros_like(accp)
            accc[...] = jnp.zeros_like(accc)

        bids = bat_ref[0, 0, :]
        P = (bids[None, :] ==
             lax.broadcasted_iota(jnp.int32, (G, NBT), 0)).astype(jnp.float32)
        hcat = jnp.concatenate([h_ref[ci] for ci in range(H // LANE)], axis=1)
        accp[...] += jnp.dot(P, hcat, preferred_element_type=jnp.float32)
        accc[...] += jnp.sum(P, axis=1, keepdims=True)

        @pl.when(i == pl.num_programs(0) - 1)
        def _():
            invg = 1.0 / jnp.maximum(accc[:, 0:1], 1.0)
            pooled = accp[...] * invg
            z = jnp.dot(pooled, w1_ref[...], preferred_element_type=jnp.float32)
            z = jnp.maximum(z + c1_ref[...], 0.0)
            z = jnp.dot(z, w2_ref[...], preferred_element_type=jnp.float32)
            z = jnp.maximum(z + c2_ref[...], 0.0)
            z = jnp.dot(z, w3_ref[...], preferred_element_type=jnp.float32)
            o_ref[...] = z + c3_ref[...]

    return pl.pallas_call(
        body,
        grid=(N // NBT,),
        in_specs=[
            pl.BlockSpec((H // LANE, NBT, LANE), lambda i: (0, i, 0)),
            pl.BlockSpec((1, 1, NBT), lambda i: (i, 0, 0)),
            pl.BlockSpec((H, H), lambda i: (0, 0)),
            pl.BlockSpec((1, H), lambda i: (0, 0)),
            pl.BlockSpec((H, H), lambda i: (0, 0)),
            pl.BlockSpec((1, H), lambda i: (0, 0)),
            pl.BlockSpec((H, C), lambda i: (0, 0)),
            pl.BlockSpec((1, C), lambda i: (0, 0)),
        ],
        out_specs=pl.BlockSpec((G, C), lambda i: (0, 0)),
        out_shape=jax.ShapeDtypeStruct((G, C), jnp.float32),
        scratch_shapes=[
            pltpu.VMEM((G, H), jnp.float32),
            pltpu.VMEM((G, LANE), jnp.float32),
        ],
    )


def kernel(x, edge_index, batch, W1r, W1s, b1, W2r, W2s, b2, W3r, W3s, b3,
           Wl1, bl1, Wl2, bl2, Wl, bl):
    src = edge_index[0]
    dst = edge_index[1]
    srcp = src
    dst2p = jnp.pad(dst.reshape(NTILES, NB, B),
                    ((0, 0), (0, NBP - NB), (0, 0)),
                    constant_values=N).reshape(NTILES * NBP, B)
    dstc = jnp.pad(dst.reshape(32, ECT), ((0, 0), (0, ECTP - ECT)),
                   constant_values=N).reshape(32 * NBC, B)
    x3 = x.reshape(N, 2, LANE).transpose(1, 0, 2)
    zrows = jnp.zeros((RPT, LANE), jnp.float32)
    ones_c = jnp.ones((B, LANE), jnp.float32)
    cnt = _make_sc_cnt()(dstc, ones_c, zrows)
    xw1 = _make_tc_xw(2)(x3, W1s.T, b1.reshape(1, H))
    agg1 = _make_sc_agg(2)(x3, srcp, dst2p, zrows)
    h1 = _make_tc_comb(2, True)(agg1, xw1, cnt, W1r.T)
    xw2 = _make_tc_xw(4)(h1, W2s.T, b2.reshape(1, H))
    agg2 = _make_sc_agg(4)(h1, srcp, dst2p, zrows)
    h2 = _make_tc_comb(4, True)(agg2, xw2, cnt, W2r.T)
    xw3 = _make_tc_xw(4)(h2, W3s.T, b3.reshape(1, H))
    agg3 = _make_sc_agg(4)(h2, srcp, dst2p, zrows)
    h3 = _make_tc_comb(4, False)(agg3, xw3, cnt, W3r.T)

    out = _make_tc_final()(
        h3, batch.reshape(N // NBT, 1, NBT),
        Wl1.T, bl1.reshape(1, H),
        Wl2.T, bl2.reshape(1, H),
        Wl.T, bl.reshape(1, C))
    return out


# speedup vs baseline: 1.0072x; 1.0072x over previous
import functools

import jax
import jax.numpy as jnp
from jax import lax
from jax.experimental import pallas as pl
from jax.experimental.pallas import tpu as pltpu
from jax.experimental.pallas import tpu_sc as plsc

N = 10000
NP = 10240
E = 160000
G = 64
C = 16
H = 512
LANE = 128
NTILES = 16
EPT = E // NTILES
EPTP = 10000
B = 80
NB = EPTP // B
NBP = 128
KH = NB // 2
RPT = NP // NTILES
ECT = E // 32
ECTP = 5120
NBC = ECTP // B
NBT = 1000


def _sc_mesh():
    return plsc.VectorSubcoreMesh(core_axis_name="c", subcore_axis_name="s")


def _make_sc_agg(nchunk):
    cpc = nchunk // 2

    def body(x3, srcr, dst2p, zrows, out,
             src_v, dst_v, rows_v, acc_s, sem):
        cid = lax.axis_index("c")
        sid = lax.axis_index("s")
        pltpu.sync_copy(srcr.at[pl.ds(sid * EPTP, EPTP)], src_v)
        pltpu.sync_copy(dst2p.at[pl.ds(sid * NBP, NBP)], dst_v)
        r0 = sid * RPT

        def run_chunk(ci):
            table = x3.at[ci]
            pltpu.sync_copy(zrows, acc_s.at[pl.ds(r0, RPT)])
            plsc.subcore_barrier()

            def step(b, carry):
                off = pl.multiple_of(b * B, B)
                pltpu.async_copy(
                    table.at[src_v.at[pl.ds(off, B)]], rows_v, sem
                ).wait()
                pltpu.sync_copy(rows_v, acc_s.at[dst_v.at[b]], add=True)
                return carry

            lax.fori_loop(0, NB, step, 0)
            plsc.subcore_barrier()
            pltpu.sync_copy(acc_s.at[pl.ds(r0, RPT)],
                            out.at[ci].at[pl.ds(r0, RPT)])
            plsc.subcore_barrier()

        @pl.when(cid == 0)
        def _():
            for ci in range(cpc):
                run_chunk(ci)

        @pl.when(cid == 1)
        def _():
            for ci in range(cpc, nchunk):
                run_chunk(ci)

    return pl.kernel(
        body,
        out_type=jax.ShapeDtypeStruct((nchunk, NP, LANE), jnp.float32),
        mesh=_sc_mesh(),
        scratch_types=[
            pltpu.VMEM((EPTP,), jnp.int32),
            pltpu.VMEM((NBP, B), jnp.int32),
            pltpu.VMEM((B, LANE), jnp.float32),
            pltpu.VMEM_SHARED((NP, LANE), jnp.float32),
            pltpu.SemaphoreType.DMA,
        ],
    )


def _make_sc_cnt():
    def body(dstc, ones_h, zrows, out, dst_v, ones_v, acc_s):
        cid = lax.axis_index("c")
        sid = lax.axis_index("s")
        r0 = sid * RPT
        w = cid * NTILES + sid
        pltpu.sync_copy(ones_h, ones_v)
        pltpu.sync_copy(dstc.at[pl.ds(w * NBC, NBC)], dst_v)
        pltpu.sync_copy(zrows, acc_s.at[pl.ds(r0, RPT)])
        plsc.subcore_barrier()

        def step(b, carry):
            pltpu.sync_copy(ones_v, acc_s.at[dst_v.at[b]], add=True)
            return carry

        lax.fori_loop(0, NBC, step, 0)
        plsc.subcore_barrier()
        pltpu.sync_copy(acc_s.at[pl.ds(r0, RPT)],
                        out.at[cid].at[pl.ds(r0, RPT)])

    return pl.kernel(
        body,
        out_type=jax.ShapeDtypeStruct((2, NP, LANE), jnp.float32),
        mesh=_sc_mesh(),
        scratch_types=[
            pltpu.VMEM((NBC, B), jnp.int32),
            pltpu.VMEM((B, LANE), jnp.float32),
            pltpu.VMEM_SHARED((NP, LANE), jnp.float32),
        ],
    )


def _make_tc_layer(nc_in, relu):
    def body(agg_ref, x_ref, cnt_ref, w_ref, b_ref, o_ref):
        cnt = cnt_ref[0, :, 0:1] + cnt_ref[1, :, 0:1]
        inv = 1.0 / jnp.maximum(cnt, 1.0)
        parts = [agg_ref[ci] * inv for ci in range(nc_in)]
        parts += [x_ref[ci] for ci in range(nc_in)]
        cat = jnp.concatenate(parts, axis=1)
        acc = jnp.dot(cat, w_ref[...], preferred_element_type=jnp.float32)
        acc = acc + b_ref[...]
        if relu:
            acc = jnp.maximum(acc, 0.0)
        for co in range(H // LANE):
            o_ref[co] = acc[:, co * LANE:(co + 1) * LANE]

    return pl.pallas_call(
        body,
        grid=(N // NBT,),
        in_specs=[
            pl.BlockSpec((nc_in, NBT, LANE), lambda i: (0, i, 0)),
            pl.BlockSpec((nc_in, NBT, LANE), lambda i: (0, i, 0)),
            pl.BlockSpec((2, NBT, LANE), lambda i: (0, i, 0)),
            pl.BlockSpec((2 * nc_in * LANE, H), lambda i: (0, 0)),
            pl.BlockSpec((1, H), lambda i: (0, 0)),
        ],
        out_specs=pl.BlockSpec((H // LANE, NBT, LANE), lambda i: (0, i, 0)),
        out_shape=jax.ShapeDtypeStruct((H // LANE, N, LANE), jnp.float32),
    )


def _make_tc_final():
    def body(h_ref, bat_ref, w1_ref, c1_ref, w2_ref, c2_ref, w3_ref, c3_ref,
             o_ref, accp, accc):
        i = pl.program_id(0)

        @pl.when(i == 0)
        def _():
            accp[...] = jnp.zeros_like(accp)
            accc[...] = jnp.zeros_like(accc)

        bids = bat_ref[0, 0, :]
        P = (bids[None, :] ==
             lax.broadcasted_iota(jnp.int32, (G, NBT), 0)).astype(jnp.float32)
        hcat = jnp.concatenate([h_ref[ci] for ci in range(H // LANE)], axis=1)
        accp[...] += jnp.dot(P, hcat, preferred_element_type=jnp.float32)
        accc[...] += jnp.sum(P, axis=1, keepdims=True)

        @pl.when(i == pl.num_programs(0) - 1)
        def _():
            invg = 1.0 / jnp.maximum(accc[:, 0:1], 1.0)
            pooled = accp[...] * invg
            z = jnp.dot(pooled, w1_ref[...], preferred_element_type=jnp.float32)
            z = jnp.maximum(z + c1_ref[...], 0.0)
            z = jnp.dot(z, w2_ref[...], preferred_element_type=jnp.float32)
            z = jnp.maximum(z + c2_ref[...], 0.0)
            z = jnp.dot(z, w3_ref[...], preferred_element_type=jnp.float32)
            o_ref[...] = z + c3_ref[...]

    return pl.pallas_call(
        body,
        grid=(N // NBT,),
        in_specs=[
            pl.BlockSpec((H // LANE, NBT, LANE), lambda i: (0, i, 0)),
            pl.BlockSpec((1, 1, NBT), lambda i: (i, 0, 0)),
            pl.BlockSpec((H, H), lambda i: (0, 0)),
            pl.BlockSpec((1, H), lambda i: (0, 0)),
            pl.BlockSpec((H, H), lambda i: (0, 0)),
            pl.BlockSpec((1, H), lambda i: (0, 0)),
            pl.BlockSpec((H, C), lambda i: (0, 0)),
            pl.BlockSpec((1, C), lambda i: (0, 0)),
        ],
        out_specs=pl.BlockSpec((G, C), lambda i: (0, 0)),
        out_shape=jax.ShapeDtypeStruct((G, C), jnp.float32),
        scratch_shapes=[
            pltpu.VMEM((G, H), jnp.float32),
            pltpu.VMEM((G, LANE), jnp.float32),
        ],
    )


def kernel(x, edge_index, batch, W1r, W1s, b1, W2r, W2s, b2, W3r, W3s, b3,
           Wl1, bl1, Wl2, bl2, Wl, bl):
    src = edge_index[0]
    dst = edge_index[1]
    srcp = src
    dst2p = jnp.pad(dst.reshape(NTILES, NB, B),
                    ((0, 0), (0, NBP - NB), (0, 0)),
                    constant_values=N).reshape(NTILES * NBP, B)
    dstc = jnp.pad(dst.reshape(32, ECT), ((0, 0), (0, ECTP - ECT)),
                   constant_values=N).reshape(32 * NBC, B)
    x3 = x.reshape(N, 2, LANE).transpose(1, 0, 2)
    zrows = jnp.zeros((RPT, LANE), jnp.float32)
    ones_c = jnp.ones((B, LANE), jnp.float32)
    W21 = jnp.concatenate([W1r.T, W1s.T], axis=0)
    W22 = jnp.concatenate([W2r.T, W2s.T], axis=0)
    W23 = jnp.concatenate([W3r.T, W3s.T], axis=0)

    cnt = _make_sc_cnt()(dstc, ones_c, zrows)
    agg1 = _make_sc_agg(2)(x3, srcp, dst2p, zrows)
    h1 = _make_tc_layer(2, True)(agg1, x3, cnt, W21, b1.reshape(1, H))
    agg2 = _make_sc_agg(4)(h1, srcp, dst2p, zrows)
    h2 = _make_tc_layer(4, True)(agg2, h1, cnt, W22, b2.reshape(1, H))
    agg3 = _make_sc_agg(4)(h2, srcp, dst2p, zrows)
    h3 = _make_tc_layer(4, False)(agg3, h2, cnt, W23, b3.reshape(1, H))

    out = _make_tc_final()(
        h3, batch.reshape(N // NBT, 1, NBT),
        Wl1.T, bl1.reshape(1, H),
        Wl2.T, bl2.reshape(1, H),
        Wl.T, bl.reshape(1, C))
    return out


# speedup vs baseline: 1.0216x; 1.0142x over previous
import functools

import jax
import jax.numpy as jnp
from jax import lax
from jax.experimental import pallas as pl
from jax.experimental.pallas import tpu as pltpu
from jax.experimental.pallas import tpu_sc as plsc

N = 10000
NP = 10240
E = 160000
G = 64
C = 16
H = 512
LANE = 128
NTILES = 16
EPT = E // NTILES
EPTP = 10000
B = 80
NB = EPTP // B
NBP = 128
KH = NB // 2
RPT = NP // NTILES
ECT = E // 32
ECTP = 5120
NBC = ECTP // B
NBT = 1000


def _sc_mesh():
    return plsc.VectorSubcoreMesh(core_axis_name="c", subcore_axis_name="s")


def _make_sc_agg(nchunk):
    cpc = nchunk // 2

    def body(x3, srcr, dst2p, zrows, out,
             src_v, dst_v, rows_v, acc_s, sem):
        cid = lax.axis_index("c")
        sid = lax.axis_index("s")
        pltpu.sync_copy(srcr.at[pl.ds(sid * EPTP, EPTP)], src_v)
        pltpu.sync_copy(dst2p.at[pl.ds(sid * NBP, NBP)], dst_v)
        r0 = sid * RPT

        def run_chunk(ci):
            table = x3.at[ci]
            pltpu.sync_copy(zrows, acc_s.at[pl.ds(r0, RPT)])
            plsc.subcore_barrier()

            def step(b, carry):
                off = pl.multiple_of(b * B, B)
                pltpu.async_copy(
                    table.at[src_v.at[pl.ds(off, B)]], rows_v, sem
                ).wait()
                pltpu.sync_copy(rows_v, acc_s.at[dst_v.at[b]], add=True)
                return carry

            lax.fori_loop(0, NB, step, 0)
            plsc.subcore_barrier()
            pltpu.sync_copy(acc_s.at[pl.ds(r0, RPT)],
                            out.at[ci].at[pl.ds(r0, RPT)])
            plsc.subcore_barrier()

        @pl.when(cid == 0)
        def _():
            for ci in range(cpc):
                run_chunk(ci)

        @pl.when(cid == 1)
        def _():
            for ci in range(cpc, nchunk):
                run_chunk(ci)

    return pl.kernel(
        body,
        out_type=jax.ShapeDtypeStruct((nchunk, NP, LANE), jnp.float32),
        mesh=_sc_mesh(),
        scratch_types=[
            pltpu.VMEM((EPTP,), jnp.int32),
            pltpu.VMEM((NBP, B), jnp.int32),
            pltpu.VMEM((B, LANE), jnp.float32),
            pltpu.VMEM_SHARED((NP, LANE), jnp.float32),
            pltpu.SemaphoreType.DMA,
        ],
    )


def _make_sc_cnt():
    def body(dstc, ones_h, zrows, out, dst_v, ones_v, acc_s):
        cid = lax.axis_index("c")
        sid = lax.axis_index("s")
        r0 = sid * RPT
        w = cid * NTILES + sid
        pltpu.sync_copy(ones_h, ones_v)
        pltpu.sync_copy(dstc.at[pl.ds(w * NBC, NBC)], dst_v)
        pltpu.sync_copy(zrows, acc_s.at[pl.ds(r0, RPT)])
        plsc.subcore_barrier()

        def step(b, carry):
            pltpu.sync_copy(ones_v, acc_s.at[dst_v.at[b]], add=True)
            return carry

        lax.fori_loop(0, NBC, step, 0)
        plsc.subcore_barrier()
        pltpu.sync_copy(acc_s.at[pl.ds(r0, RPT)],
                        out.at[cid].at[pl.ds(r0, RPT)])

    return pl.kernel(
        body,
        out_type=jax.ShapeDtypeStruct((2, NP, LANE), jnp.float32),
        mesh=_sc_mesh(),
        scratch_types=[
            pltpu.VMEM((NBC, B), jnp.int32),
            pltpu.VMEM((B, LANE), jnp.float32),
            pltpu.VMEM_SHARED((NP, LANE), jnp.float32),
        ],
    )


def _make_tc_layer(nc_in, relu):
    def body(agg_ref, x_ref, cnt_ref, w_ref, b_ref, o_ref):
        cnt = cnt_ref[0, :, 0:1] + cnt_ref[1, :, 0:1]
        inv = 1.0 / jnp.maximum(cnt, 1.0)
        parts = [agg_ref[ci] * inv for ci in range(nc_in)]
        parts += [x_ref[ci] for ci in range(nc_in)]
        cat = jnp.concatenate(parts, axis=1)
        acc = jnp.dot(cat, w_ref[...], preferred_element_type=jnp.float32)
        acc = acc + b_ref[...]
        if relu:
            acc = jnp.maximum(acc, 0.0)
        for co in range(H // LANE):
            o_ref[co] = acc[:, co * LANE:(co + 1) * LANE]

    return pl.pallas_call(
        body,
        grid=(N // NBT,),
        in_specs=[
            pl.BlockSpec((nc_in, NBT, LANE), lambda i: (0, i, 0)),
            pl.BlockSpec((nc_in, NBT, LANE), lambda i: (0, i, 0)),
            pl.BlockSpec((2, NBT, LANE), lambda i: (0, i, 0)),
            pl.BlockSpec((2 * nc_in * LANE, H), lambda i: (0, 0)),
            pl.BlockSpec((1, H), lambda i: (0, 0)),
        ],
        out_specs=pl.BlockSpec((H // LANE, NBT, LANE), lambda i: (0, i, 0)),
        out_shape=jax.ShapeDtypeStruct((H // LANE, N, LANE), jnp.float32),
    )


def _make_tc_last():
    def body(agg_ref, x_ref, cnt_ref, w_ref, b_ref, bat_ref,
             w1_ref, c1_ref, w2_ref, c2_ref, w3_ref, c3_ref,
             o_ref, accp, accc):
        i = pl.program_id(0)

        @pl.when(i == 0)
        def _():
            accp[...] = jnp.zeros_like(accp)
            accc[...] = jnp.zeros_like(accc)

        cnt = cnt_ref[0, :, 0:1] + cnt_ref[1, :, 0:1]
        inv = 1.0 / jnp.maximum(cnt, 1.0)
        parts = [agg_ref[ci] * inv for ci in range(4)]
        parts += [x_ref[ci] for ci in range(4)]
        cat = jnp.concatenate(parts, axis=1)
        h = jnp.dot(cat, w_ref[...], preferred_element_type=jnp.float32)
        h = h + b_ref[...]

        bids = bat_ref[0, 0, :]
        P = (bids[None, :] ==
             lax.broadcasted_iota(jnp.int32, (G, NBT), 0)).astype(jnp.float32)
        accp[...] += jnp.dot(P, h, preferred_element_type=jnp.float32)
        accc[...] += jnp.sum(P, axis=1, keepdims=True)

        @pl.when(i == pl.num_programs(0) - 1)
        def _():
            invg = 1.0 / jnp.maximum(accc[:, 0:1], 1.0)
            pooled = accp[...] * invg
            z = jnp.dot(pooled, w1_ref[...], preferred_element_type=jnp.float32)
            z = jnp.maximum(z + c1_ref[...], 0.0)
            z = jnp.dot(z, w2_ref[...], preferred_element_type=jnp.float32)
            z = jnp.maximum(z + c2_ref[...], 0.0)
            z = jnp.dot(z, w3_ref[...], preferred_element_type=jnp.float32)
            o_ref[...] = z + c3_ref[...]

    return pl.pallas_call(
        body,
        grid=(N // NBT,),
        in_specs=[
            pl.BlockSpec((4, NBT, LANE), lambda i: (0, i, 0)),
            pl.BlockSpec((4, NBT, LANE), lambda i: (0, i, 0)),
            pl.BlockSpec((2, NBT, LANE), lambda i: (0, i, 0)),
            pl.BlockSpec((8 * LANE, H), lambda i: (0, 0)),
            pl.BlockSpec((1, H), lambda i: (0, 0)),
            pl.BlockSpec((1, 1, NBT), lambda i: (i, 0, 0)),
            pl.BlockSpec((H, H), lambda i: (0, 0)),
            pl.BlockSpec((1, H), lambda i: (0, 0)),
            pl.BlockSpec((H, H), lambda i: (0, 0)),
            pl.BlockSpec((1, H), lambda i: (0, 0)),
            pl.BlockSpec((H, C), lambda i: (0, 0)),
            pl.BlockSpec((1, C), lambda i: (0, 0)),
        ],
        out_specs=pl.BlockSpec((G, C), lambda i: (0, 0)),
        out_shape=jax.ShapeDtypeStruct((G, C), jnp.float32),
        scratch_shapes=[
            pltpu.VMEM((G, H), jnp.float32),
            pltpu.VMEM((G, LANE), jnp.float32),
        ],
    )


def _make_tc_final():
    def body(h_ref, bat_ref, w1_ref, c1_ref, w2_ref, c2_ref, w3_ref, c3_ref,
             o_ref, accp, accc):
        i = pl.program_id(0)

        @pl.when(i == 0)
        def _():
            accp[...] = jnp.zeros_like(accp)
            accc[...] = jnp.zeros_like(accc)

        bids = bat_ref[0, 0, :]
        P = (bids[None, :] ==
             lax.broadcasted_iota(jnp.int32, (G, NBT), 0)).astype(jnp.float32)
        hcat = jnp.concatenate([h_ref[ci] for ci in range(H // LANE)], axis=1)
        accp[...] += jnp.dot(P, hcat, preferred_element_type=jnp.float32)
        accc[...] += jnp.sum(P, axis=1, keepdims=True)

        @pl.when(i == pl.num_programs(0) - 1)
        def _():
            invg = 1.0 / jnp.maximum(accc[:, 0:1], 1.0)
            pooled = accp[...] * invg
            z = jnp.dot(pooled, w1_ref[...], preferred_element_type=jnp.float32)
            z = jnp.maximum(z + c1_ref[...], 0.0)
            z = jnp.dot(z, w2_ref[...], preferred_element_type=jnp.float32)
            z = jnp.maximum(z + c2_ref[...], 0.0)
            z = jnp.dot(z, w3_ref[...], preferred_element_type=jnp.float32)
            o_ref[...] = z + c3_ref[...]

    return pl.pallas_call(
        body,
        grid=(N // NBT,),
        in_specs=[
            pl.BlockSpec((H // LANE, NBT, LANE), lambda i: (0, i, 0)),
            pl.BlockSpec((1, 1, NBT), lambda i: (i, 0, 0)),
            pl.BlockSpec((H, H), lambda i: (0, 0)),
            pl.BlockSpec((1, H), lambda i: (0, 0)),
            pl.BlockSpec((H, H), lambda i: (0, 0)),
            pl.BlockSpec((1, H), lambda i: (0, 0)),
            pl.BlockSpec((H, C), lambda i: (0, 0)),
            pl.BlockSpec((1, C), lambda i: (0, 0)),
        ],
        out_specs=pl.BlockSpec((G, C), lambda i: (0, 0)),
        out_shape=jax.ShapeDtypeStruct((G, C), jnp.float32),
        scratch_shapes=[
            pltpu.VMEM((G, H), jnp.float32),
            pltpu.VMEM((G, LANE), jnp.float32),
        ],
    )


def kernel(x, edge_index, batch, W1r, W1s, b1, W2r, W2s, b2, W3r, W3s, b3,
           Wl1, bl1, Wl2, bl2, Wl, bl):
    src = edge_index[0]
    dst = edge_index[1]
    srcp = src
    dst2p = jnp.pad(dst.reshape(NTILES, NB, B),
                    ((0, 0), (0, NBP - NB), (0, 0)),
                    constant_values=N).reshape(NTILES * NBP, B)
    dstc = jnp.pad(dst.reshape(32, ECT), ((0, 0), (0, ECTP - ECT)),
                   constant_values=N).reshape(32 * NBC, B)
    x3 = x.reshape(N, 2, LANE).transpose(1, 0, 2)
    zrows = jnp.zeros((RPT, LANE), jnp.float32)
    ones_c = jnp.ones((B, LANE), jnp.float32)
    W21 = jnp.concatenate([W1r.T, W1s.T], axis=0)
    W22 = jnp.concatenate([W2r.T, W2s.T], axis=0)
    W23 = jnp.concatenate([W3r.T, W3s.T], axis=0)

    cnt = _make_sc_cnt()(dstc, ones_c, zrows)
    agg1 = _make_sc_agg(2)(x3, srcp, dst2p, zrows)
    h1 = _make_tc_layer(2, True)(agg1, x3, cnt, W21, b1.reshape(1, H))
    agg2 = _make_sc_agg(4)(h1, srcp, dst2p, zrows)
    h2 = _make_tc_layer(4, True)(agg2, h1, cnt, W22, b2.reshape(1, H))
    agg3 = _make_sc_agg(4)(h2, srcp, dst2p, zrows)
    out = _make_tc_last()(
        agg3, h2, cnt, W23, b3.reshape(1, H),
        batch.reshape(N // NBT, 1, NBT),
        Wl1.T, bl1.reshape(1, H),
        Wl2.T, bl2.reshape(1, H),
        Wl.T, bl.reshape(1, C))
    return out


# speedup vs baseline: 1.0242x; 1.0026x over previous
import functools

import jax
import jax.numpy as jnp
from jax import lax
from jax.experimental import pallas as pl
from jax.experimental.pallas import tpu as pltpu
from jax.experimental.pallas import tpu_sc as plsc

N = 10000
NP = 10240
E = 160000
G = 64
C = 16
H = 512
LANE = 128
NTILES = 16
EPT = E // NTILES
EPTP = 10000
B = 80
NB = EPTP // B
NBP = 128
KH = NB // 2
RPT = NP // NTILES
ECT = E // 32
ECTP = 5120
NBC = ECTP // B
NBT = 1000


def _sc_mesh():
    return plsc.VectorSubcoreMesh(core_axis_name="c", subcore_axis_name="s")


def _make_sc_agg(nchunk, with_cnt=False):
    cpc = nchunk // 2

    def body(x3, srcr, dst2p, dstc, ones_h, zrows, out,
             src_v, dst_v, dstc_v, rows_v, acc_s, sem):
        cid = lax.axis_index("c")
        sid = lax.axis_index("s")
        pltpu.sync_copy(srcr.at[pl.ds(sid * EPTP, EPTP)], src_v)
        pltpu.sync_copy(dst2p.at[pl.ds(sid * NBP, NBP)], dst_v)
        r0 = sid * RPT

        def run_chunk(ci):
            table = x3.at[ci]
            pltpu.sync_copy(zrows, acc_s.at[pl.ds(r0, RPT)])
            plsc.subcore_barrier()

            def step(b, carry):
                off = pl.multiple_of(b * B, B)
                pltpu.async_copy(
                    table.at[src_v.at[pl.ds(off, B)]], rows_v, sem
                ).wait()
                pltpu.sync_copy(rows_v, acc_s.at[dst_v.at[b]], add=True)
                return carry

            lax.fori_loop(0, NB, step, 0)
            plsc.subcore_barrier()
            pltpu.sync_copy(acc_s.at[pl.ds(r0, RPT)],
                            out.at[ci].at[pl.ds(r0, RPT)])
            plsc.subcore_barrier()

        @pl.when(cid == 0)
        def _():
            for ci in range(cpc):
                run_chunk(ci)

        @pl.when(cid == 1)
        def _():
            for ci in range(cpc, nchunk):
                run_chunk(ci)

        if with_cnt:
            w = cid * NTILES + sid
            pltpu.sync_copy(ones_h, rows_v)
            pltpu.sync_copy(dstc.at[pl.ds(w * NBC, NBC)], dstc_v)
            pltpu.sync_copy(zrows, acc_s.at[pl.ds(r0, RPT)])
            plsc.subcore_barrier()

            def cstep(b, carry):
                pltpu.sync_copy(rows_v, acc_s.at[dstc_v.at[b]], add=True)
                return carry

            lax.fori_loop(0, NBC, cstep, 0)
            plsc.subcore_barrier()
            pltpu.sync_copy(acc_s.at[pl.ds(r0, RPT)],
                            out.at[nchunk + cid].at[pl.ds(r0, RPT)])

    nout = nchunk + 2 if with_cnt else nchunk
    return pl.kernel(
        body,
        out_type=jax.ShapeDtypeStruct((nout, NP, LANE), jnp.float32),
        mesh=_sc_mesh(),
        scratch_types=[
            pltpu.VMEM((EPTP,), jnp.int32),
            pltpu.VMEM((NBP, B), jnp.int32),
            pltpu.VMEM((NBC, B), jnp.int32),
            pltpu.VMEM((B, LANE), jnp.float32),
            pltpu.VMEM_SHARED((NP, LANE), jnp.float32),
            pltpu.SemaphoreType.DMA,
        ],
    )


def _make_tc_layer(nc_in, relu):
    def body(agg_ref, x_ref, cnt_ref, w_ref, b_ref, o_ref):
        cnt = cnt_ref[0, :, 0:1] + cnt_ref[1, :, 0:1]
        inv = 1.0 / jnp.maximum(cnt, 1.0)
        parts = [agg_ref[ci] * inv for ci in range(nc_in)]
        parts += [x_ref[ci] for ci in range(nc_in)]
        cat = jnp.concatenate(parts, axis=1)
        acc = jnp.dot(cat, w_ref[...], preferred_element_type=jnp.float32)
        acc = acc + b_ref[...]
        if relu:
            acc = jnp.maximum(acc, 0.0)
        for co in range(H // LANE):
            o_ref[co] = acc[:, co * LANE:(co + 1) * LANE]

    return pl.pallas_call(
        body,
        grid=(N // NBT,),
        in_specs=[
            pl.BlockSpec((nc_in, NBT, LANE), lambda i: (0, i, 0)),
            pl.BlockSpec((nc_in, NBT, LANE), lambda i: (0, i, 0)),
            pl.BlockSpec((2, NBT, LANE), lambda i: (1, i, 0)),
            pl.BlockSpec((2 * nc_in * LANE, H), lambda i: (0, 0)),
            pl.BlockSpec((1, H), lambda i: (0, 0)),
        ],
        out_specs=pl.BlockSpec((H // LANE, NBT, LANE), lambda i: (0, i, 0)),
        out_shape=jax.ShapeDtypeStruct((H // LANE, N, LANE), jnp.float32),
    )


def _make_tc_last():
    def body(agg_ref, x_ref, cnt_ref, w_ref, b_ref, bat_ref,
             w1_ref, c1_ref, w2_ref, c2_ref, w3_ref, c3_ref,
             o_ref, accp, accc):
        i = pl.program_id(0)

        @pl.when(i == 0)
        def _():
            accp[...] = jnp.zeros_like(accp)
            accc[...] = jnp.zeros_like(accc)

        cnt = cnt_ref[0, :, 0:1] + cnt_ref[1, :, 0:1]
        inv = 1.0 / jnp.maximum(cnt, 1.0)
        parts = [agg_ref[ci] * inv for ci in range(4)]
        parts += [x_ref[ci] for ci in range(4)]
        cat = jnp.concatenate(parts, axis=1)
        h = jnp.dot(cat, w_ref[...], preferred_element_type=jnp.float32)
        h = h + b_ref[...]

        bids = bat_ref[0, 0, :]
        P = (bids[None, :] ==
             lax.broadcasted_iota(jnp.int32, (G, NBT), 0)).astype(jnp.float32)
        accp[...] += jnp.dot(P, h, preferred_element_type=jnp.float32)
        accc[...] += jnp.sum(P, axis=1, keepdims=True)

        @pl.when(i == pl.num_programs(0) - 1)
        def _():
            invg = 1.0 / jnp.maximum(accc[:, 0:1], 1.0)
            pooled = accp[...] * invg
            z = jnp.dot(pooled, w1_ref[...], preferred_element_type=jnp.float32)
            z = jnp.maximum(z + c1_ref[...], 0.0)
            z = jnp.dot(z, w2_ref[...], preferred_element_type=jnp.float32)
            z = jnp.maximum(z + c2_ref[...], 0.0)
            z = jnp.dot(z, w3_ref[...], preferred_element_type=jnp.float32)
            o_ref[...] = z + c3_ref[...]

    return pl.pallas_call(
        body,
        grid=(N // NBT,),
        in_specs=[
            pl.BlockSpec((4, NBT, LANE), lambda i: (0, i, 0)),
            pl.BlockSpec((4, NBT, LANE), lambda i: (0, i, 0)),
            pl.BlockSpec((2, NBT, LANE), lambda i: (1, i, 0)),
            pl.BlockSpec((8 * LANE, H), lambda i: (0, 0)),
            pl.BlockSpec((1, H), lambda i: (0, 0)),
            pl.BlockSpec((1, 1, NBT), lambda i: (i, 0, 0)),
            pl.BlockSpec((H, H), lambda i: (0, 0)),
            pl.BlockSpec((1, H), lambda i: (0, 0)),
            pl.BlockSpec((H, H), lambda i: (0, 0)),
            pl.BlockSpec((1, H), lambda i: (0, 0)),
            pl.BlockSpec((H, C), lambda i: (0, 0)),
            pl.BlockSpec((1, C), lambda i: (0, 0)),
        ],
        out_specs=pl.BlockSpec((G, C), lambda i: (0, 0)),
        out_shape=jax.ShapeDtypeStruct((G, C), jnp.float32),
        scratch_shapes=[
            pltpu.VMEM((G, H), jnp.float32),
            pltpu.VMEM((G, LANE), jnp.float32),
        ],
    )


def _make_tc_final():
    def body(h_ref, bat_ref, w1_ref, c1_ref, w2_ref, c2_ref, w3_ref, c3_ref,
             o_ref, accp, accc):
        i = pl.program_id(0)

        @pl.when(i == 0)
        def _():
            accp[...] = jnp.zeros_like(accp)
            accc[...] = jnp.zeros_like(accc)

        bids = bat_ref[0, 0, :]
        P = (bids[None, :] ==
             lax.broadcasted_iota(jnp.int32, (G, NBT), 0)).astype(jnp.float32)
        hcat = jnp.concatenate([h_ref[ci] for ci in range(H // LANE)], axis=1)
        accp[...] += jnp.dot(P, hcat, preferred_element_type=jnp.float32)
        accc[...] += jnp.sum(P, axis=1, keepdims=True)

        @pl.when(i == pl.num_programs(0) - 1)
        def _():
            invg = 1.0 / jnp.maximum(accc[:, 0:1], 1.0)
            pooled = accp[...] * invg
            z = jnp.dot(pooled, w1_ref[...], preferred_element_type=jnp.float32)
            z = jnp.maximum(z + c1_ref[...], 0.0)
            z = jnp.dot(z, w2_ref[...], preferred_element_type=jnp.float32)
            z = jnp.maximum(z + c2_ref[...], 0.0)
            z = jnp.dot(z, w3_ref[...], preferred_element_type=jnp.float32)
            o_ref[...] = z + c3_ref[...]

    return pl.pallas_call(
        body,
        grid=(N // NBT,),
        in_specs=[
            pl.BlockSpec((H // LANE, NBT, LANE), lambda i: (0, i, 0)),
            pl.BlockSpec((1, 1, NBT), lambda i: (i, 0, 0)),
            pl.BlockSpec((H, H), lambda i: (0, 0)),
            pl.BlockSpec((1, H), lambda i: (0, 0)),
            pl.BlockSpec((H, H), lambda i: (0, 0)),
            pl.BlockSpec((1, H), lambda i: (0, 0)),
            pl.BlockSpec((H, C), lambda i: (0, 0)),
            pl.BlockSpec((1, C), lambda i: (0, 0)),
        ],
        out_specs=pl.BlockSpec((G, C), lambda i: (0, 0)),
        out_shape=jax.ShapeDtypeStruct((G, C), jnp.float32),
        scratch_shapes=[
            pltpu.VMEM((G, H), jnp.float32),
            pltpu.VMEM((G, LANE), jnp.float32),
        ],
    )


def kernel(x, edge_index, batch, W1r, W1s, b1, W2r, W2s, b2, W3r, W3s, b3,
           Wl1, bl1, Wl2, bl2, Wl, bl):
    src = edge_index[0]
    dst = edge_index[1]
    srcp = src
    dst2p = jnp.pad(dst.reshape(NTILES, NB, B),
                    ((0, 0), (0, NBP - NB), (0, 0)),
                    constant_values=N).reshape(NTILES * NBP, B)
    dstc = jnp.pad(dst.reshape(32, ECT), ((0, 0), (0, ECTP - ECT)),
                   constant_values=N).reshape(32 * NBC, B)
    x3 = x.reshape(N, 2, LANE).transpose(1, 0, 2)
    zrows = jnp.zeros((RPT, LANE), jnp.float32)
    ones_c = jnp.ones((B, LANE), jnp.float32)
    W21 = jnp.concatenate([W1r.T, W1s.T], axis=0)
    W22 = jnp.concatenate([W2r.T, W2s.T], axis=0)
    W23 = jnp.concatenate([W3r.T, W3s.T], axis=0)

    agg1c = _make_sc_agg(2, with_cnt=True)(x3, srcp, dst2p, dstc, ones_c,
                                           zrows)
    h1 = _make_tc_layer(2, True)(agg1c, x3, agg1c, W21, b1.reshape(1, H))
    agg2 = _make_sc_agg(4)(h1, srcp, dst2p, dstc, ones_c, zrows)
    h2 = _make_tc_layer(4, True)(agg2, h1, agg1c, W22, b2.reshape(1, H))
    agg3 = _make_sc_agg(4)(h2, srcp, dst2p, dstc, ones_c, zrows)
    out = _make_tc_last()(
        agg3, h2, agg1c, W23, b3.reshape(1, H),
        batch.reshape(N // NBT, 1, NBT),
        Wl1.T, bl1.reshape(1, H),
        Wl2.T, bl2.reshape(1, H),
        Wl.T, bl.reshape(1, C))
    return out
